# baseline (speedup 1.0000x reference)
import jax
import jax.numpy as jnp
from jax import lax
from jax.experimental import pallas as pl
from jax.experimental.pallas import tpu as pltpu

N, E, D, H, C = 10000, 160000, 256, 256, 16
NP = 10240
BN = 1024
NB = NP // BN
EP = 163840
CE = 8192
NG = EP // CE


def _tc_proj1(x_ref, wl_ref, wr_ref, b_ref, p_ref, q_ref):
    p_ref[...] = jnp.dot(x_ref[...], wl_ref[...],
                         preferred_element_type=jnp.float32)
    q_ref[...] = (
        jnp.dot(x_ref[...], wr_ref[...], preferred_element_type=jnp.float32)
        + b_ref[0:1, :]
    )


def _tc_agg(src_ref, dst_ref, p_ref, agg_ref, deg_ref):
    @pl.when(pl.program_id(0) == 0)
    def _():
        agg_ref[...] = jnp.zeros_like(agg_ref)
        deg_ref[...] = jnp.zeros_like(deg_ref)

    one8 = jnp.full((1, 8), 1.0, jnp.float32)

    def body(e, carry):
        s_ = src_ref[e]
        d_ = dst_ref[e]
        row = p_ref[pl.ds(s_, 1), :]
        agg_ref[pl.ds(d_, 1), :] += row
        deg_ref[pl.ds(d_, 1), :] += one8
        return carry

    lax.fori_loop(0, CE, body, 0)


def _tc_agg2(src_ref, dst_ref, p_ref, agg_ref):
    @pl.when(pl.program_id(0) == 0)
    def _():
        agg_ref[...] = jnp.zeros_like(agg_ref)

    def body(e, carry):
        s_ = src_ref[e]
        d_ = dst_ref[e]
        agg_ref[pl.ds(d_, 1), :] += p_ref[pl.ds(s_, 1), :]
        return carry

    lax.fori_loop(0, CE, body, 0)


def _tc_layer2(agg_ref, deg_ref, q1_ref, wl_ref, wr_ref, b_ref,
               p2_ref, q2_ref):
    rdeg = 1.0 / jnp.maximum(deg_ref[:, 0:1], 1.0)
    h = jnp.maximum(agg_ref[...] * rdeg + q1_ref[...], 0.0)
    p2_ref[...] = jnp.dot(h, wl_ref[...], preferred_element_type=jnp.float32)
    q2_ref[...] = (
        jnp.dot(h, wr_ref[...], preferred_element_type=jnp.float32)
        + b_ref[0:1, :]
    )


def _tc_softmax(agg_ref, deg_ref, q2_ref, o_ref):
    rdeg = 1.0 / jnp.maximum(deg_ref[:, 0:1], 1.0)
    s = agg_ref[...] * rdeg + q2_ref[...]
    m = jnp.max(s, axis=1, keepdims=True)
    e = jnp.exp(s - m)
    o_ref[...] = e / jnp.sum(e, axis=1, keepdims=True)


def kernel(x, edge_index, Wl1, bl1, Wr1, Wl2, bl2, Wr2):
    f32 = jnp.float32
    pad_r = jnp.arange(EP - E, dtype=jnp.int32)
    src = jnp.concatenate([edge_index[0], pad_r % N])
    dst = jnp.concatenate([edge_index[1], N + pad_r % (NP - N)])
    xp = jnp.pad(x, ((0, NP - N), (0, 0)))
    bl1b = jnp.broadcast_to(bl1, (8, H))
    bl2b = jnp.broadcast_to(bl2, (8, C))

    p1, q1 = pl.pallas_call(
        _tc_proj1,
        grid=(NB,),
        in_specs=[
            pl.BlockSpec((BN, D), lambda i: (i, 0)),
            pl.BlockSpec((D, H), lambda i: (0, 0)),
            pl.BlockSpec((D, H), lambda i: (0, 0)),
            pl.BlockSpec((8, H), lambda i: (0, 0)),
        ],
        out_specs=[
            pl.BlockSpec((BN, H), lambda i: (i, 0)),
            pl.BlockSpec((BN, H), lambda i: (i, 0)),
        ],
        out_shape=[
            jax.ShapeDtypeStruct((NP, H), f32),
            jax.ShapeDtypeStruct((NP, H), f32),
        ],
    )(xp, Wl1, Wr1, bl1b)

    agg1, deg8 = pl.pallas_call(
        _tc_agg,
        grid=(NG,),
        in_specs=[
            pl.BlockSpec((CE,), lambda i: (i,), memory_space=pltpu.SMEM),
            pl.BlockSpec((CE,), lambda i: (i,), memory_space=pltpu.SMEM),
            pl.BlockSpec((NP, H), lambda i: (0, 0)),
        ],
        out_specs=[
            pl.BlockSpec((NP, H), lambda i: (0, 0)),
            pl.BlockSpec((NP, 8), lambda i: (0, 0)),
        ],
        out_shape=[
            jax.ShapeDtypeStruct((NP, H), f32),
            jax.ShapeDtypeStruct((NP, 8), f32),
        ],
    )(src, dst, p1)

    p2, q2 = pl.pallas_call(
        _tc_layer2,
        grid=(NB,),
        in_specs=[
            pl.BlockSpec((BN, H), lambda i: (i, 0)),
            pl.BlockSpec((BN, 8), lambda i: (i, 0)),
            pl.BlockSpec((BN, H), lambda i: (i, 0)),
            pl.BlockSpec((H, C), lambda i: (0, 0)),
            pl.BlockSpec((H, C), lambda i: (0, 0)),
            pl.BlockSpec((8, C), lambda i: (0, 0)),
        ],
        out_specs=[
            pl.BlockSpec((BN, C), lambda i: (i, 0)),
            pl.BlockSpec((BN, C), lambda i: (i, 0)),
        ],
        out_shape=[
            jax.ShapeDtypeStruct((NP, C), f32),
            jax.ShapeDtypeStruct((NP, C), f32),
        ],
    )(agg1, deg8, q1, Wl2, Wr2, bl2b)

    agg2 = pl.pallas_call(
        _tc_agg2,
        grid=(NG,),
        in_specs=[
            pl.BlockSpec((CE,), lambda i: (i,), memory_space=pltpu.SMEM),
            pl.BlockSpec((CE,), lambda i: (i,), memory_space=pltpu.SMEM),
            pl.BlockSpec((NP, C), lambda i: (0, 0)),
        ],
        out_specs=pl.BlockSpec((NP, C), lambda i: (0, 0)),
        out_shape=jax.ShapeDtypeStruct((NP, C), f32),
    )(src, dst, p2)

    out = pl.pallas_call(
        _tc_softmax,
        grid=(NB,),
        in_specs=[
            pl.BlockSpec((BN, C), lambda i: (i, 0)),
            pl.BlockSpec((BN, 8), lambda i: (i, 0)),
            pl.BlockSpec((BN, C), lambda i: (i, 0)),
        ],
        out_specs=pl.BlockSpec((BN, C), lambda i: (i, 0)),
        out_shape=jax.ShapeDtypeStruct((NP, C), f32),
    )(agg2, deg8, q2)
    return out[:N]


# speedup vs baseline: 1.8702x; 1.8702x over previous
import jax
import jax.numpy as jnp
from jax import lax
from jax.experimental import pallas as pl
from jax.experimental.pallas import tpu as pltpu

N, E, D, H, C = 10000, 160000, 256, 256, 16
NP = 10240
BN = 1024
NB = NP // BN
EP = 163840
CE = 8192
NG = EP // CE


def _tc_proj1(x_ref, wl_ref, wr_ref, b_ref, p_ref, q_ref):
    p_ref[...] = jnp.dot(x_ref[...], wl_ref[...],
                         preferred_element_type=jnp.float32)
    q_ref[...] = (
        jnp.dot(x_ref[...], wr_ref[...], preferred_element_type=jnp.float32)
        + b_ref[0:1, :]
    )


def _tc_agg(src_ref, dst_ref, p_ref, agg_ref, deg_ref):
    @pl.when(pl.program_id(0) == 0)
    def _():
        agg_ref[...] = jnp.zeros_like(agg_ref)
        deg_ref[...] = jnp.zeros_like(deg_ref)

    one8 = jnp.full((1, 8), 1.0, jnp.float32)

    def body(b, carry):
        for u in range(8):
            e = b * 8 + u
            s_ = src_ref[e]
            d_ = dst_ref[e]
            agg_ref[pl.ds(d_, 1), :] += p_ref[pl.ds(s_, 1), :]
            deg_ref[pl.ds(d_, 1), :] += one8
        return carry

    lax.fori_loop(0, CE // 8, body, 0)


def _tc_agg2(src_ref, dst_ref, p_ref, agg_ref):
    @pl.when(pl.program_id(0) == 0)
    def _():
        agg_ref[...] = jnp.zeros_like(agg_ref)

    def body(b, carry):
        for u in range(8):
            e = b * 8 + u
            s_ = src_ref[e]
            d_ = dst_ref[e]
            agg_ref[pl.ds(d_, 1), :] += p_ref[pl.ds(s_, 1), :]
        return carry

    lax.fori_loop(0, CE // 8, body, 0)


def _tc_layer2(agg_ref, deg_ref, q1_ref, wl_ref, wr_ref, b_ref,
               p2_ref, q2_ref):
    rdeg = 1.0 / jnp.maximum(deg_ref[:, 0:1], 1.0)
    h = jnp.maximum(agg_ref[...] * rdeg + q1_ref[...], 0.0)
    p2_ref[...] = jnp.dot(h, wl_ref[...], preferred_element_type=jnp.float32)
    q2_ref[...] = (
        jnp.dot(h, wr_ref[...], preferred_element_type=jnp.float32)
        + b_ref[0:1, :]
    )


def _tc_softmax(agg_ref, deg_ref, q2_ref, o_ref):
    rdeg = 1.0 / jnp.maximum(deg_ref[:, 0:1], 1.0)
    s = agg_ref[...] * rdeg + q2_ref[...]
    m = jnp.max(s, axis=1, keepdims=True)
    e = jnp.exp(s - m)
    o_ref[...] = e / jnp.sum(e, axis=1, keepdims=True)


def kernel(x, edge_index, Wl1, bl1, Wr1, Wl2, bl2, Wr2):
    f32 = jnp.float32
    pad_r = jnp.arange(EP - E, dtype=jnp.int32)
    src = jnp.concatenate([edge_index[0], pad_r % N])
    dst = jnp.concatenate([edge_index[1], N + pad_r % (NP - N)])
    xp = jnp.pad(x, ((0, NP - N), (0, 0)))
    bl1b = jnp.broadcast_to(bl1, (8, H))
    bl2b = jnp.broadcast_to(bl2, (8, C))

    p1, q1 = pl.pallas_call(
        _tc_proj1,
        grid=(NB,),
        in_specs=[
            pl.BlockSpec((BN, D), lambda i: (i, 0)),
            pl.BlockSpec((D, H), lambda i: (0, 0)),
            pl.BlockSpec((D, H), lambda i: (0, 0)),
            pl.BlockSpec((8, H), lambda i: (0, 0)),
        ],
        out_specs=[
            pl.BlockSpec((BN, H), lambda i: (i, 0)),
            pl.BlockSpec((BN, H), lambda i: (i, 0)),
        ],
        out_shape=[
            jax.ShapeDtypeStruct((NP, H), f32),
            jax.ShapeDtypeStruct((NP, H), f32),
        ],
    )(xp, Wl1, Wr1, bl1b)

    agg1, deg8 = pl.pallas_call(
        _tc_agg,
        grid=(NG,),
        in_specs=[
            pl.BlockSpec((CE,), lambda i: (i,), memory_space=pltpu.SMEM),
            pl.BlockSpec((CE,), lambda i: (i,), memory_space=pltpu.SMEM),
            pl.BlockSpec((NP, H), lambda i: (0, 0)),
        ],
        out_specs=[
            pl.BlockSpec((NP, H), lambda i: (0, 0)),
            pl.BlockSpec((NP, 8), lambda i: (0, 0)),
        ],
        out_shape=[
            jax.ShapeDtypeStruct((NP, H), f32),
            jax.ShapeDtypeStruct((NP, 8), f32),
        ],
    )(src, dst, p1)

    p2, q2 = pl.pallas_call(
        _tc_layer2,
        grid=(NB,),
        in_specs=[
            pl.BlockSpec((BN, H), lambda i: (i, 0)),
            pl.BlockSpec((BN, 8), lambda i: (i, 0)),
            pl.BlockSpec((BN, H), lambda i: (i, 0)),
            pl.BlockSpec((H, C), lambda i: (0, 0)),
            pl.BlockSpec((H, C), lambda i: (0, 0)),
            pl.BlockSpec((8, C), lambda i: (0, 0)),
        ],
        out_specs=[
            pl.BlockSpec((BN, C), lambda i: (i, 0)),
            pl.BlockSpec((BN, C), lambda i: (i, 0)),
        ],
        out_shape=[
            jax.ShapeDtypeStruct((NP, C), f32),
            jax.ShapeDtypeStruct((NP, C), f32),
        ],
    )(agg1, deg8, q1, Wl2, Wr2, bl2b)

    agg2 = pl.pallas_call(
        _tc_agg2,
        grid=(NG,),
        in_specs=[
            pl.BlockSpec((CE,), lambda i: (i,), memory_space=pltpu.SMEM),
            pl.BlockSpec((CE,), lambda i: (i,), memory_space=pltpu.SMEM),
            pl.BlockSpec((NP, C), lambda i: (0, 0)),
        ],
        out_specs=pl.BlockSpec((NP, C), lambda i: (0, 0)),
        out_shape=jax.ShapeDtypeStruct((NP, C), f32),
    )(src, dst, p2)

    out = pl.pallas_call(
        _tc_softmax,
        grid=(NB,),
        in_specs=[
            pl.BlockSpec((BN, C), lambda i: (i, 0)),
            pl.BlockSpec((BN, 8), lambda i: (i, 0)),
            pl.BlockSpec((BN, C), lambda i: (i, 0)),
        ],
        out_specs=pl.BlockSpec((BN, C), lambda i: (i, 0)),
        out_shape=jax.ShapeDtypeStruct((NP, C), f32),
    )(agg2, deg8, q2)
    return out[:N]


# speedup vs baseline: 1.9901x; 1.0641x over previous
import jax
import jax.numpy as jnp
from jax import lax
from jax.experimental import pallas as pl
from jax.experimental.pallas import tpu as pltpu

N, E, D, H, C = 10000, 160000, 256, 256, 16
NP = 10240
BN = 1024
NB = NP // BN
EP = 163840
CE = 8192
NG = EP // CE


def _tc_proj1(x_ref, wl_ref, wr_ref, b_ref, p_ref, q_ref):
    p_ref[...] = jnp.dot(x_ref[...], wl_ref[...],
                         preferred_element_type=jnp.float32)
    q_ref[...] = (
        jnp.dot(x_ref[...], wr_ref[...], preferred_element_type=jnp.float32)
        + b_ref[0:1, :]
    )


def _tc_agg(src_ref, dst_ref, p_ref, agg_ref, deg_ref):
    @pl.when(pl.program_id(0) == 0)
    def _():
        agg_ref[...] = jnp.zeros_like(agg_ref)
        deg_ref[...] = jnp.zeros_like(deg_ref)

    one8 = jnp.full((1, 8), 1.0, jnp.float32)

    def body(b, carry):
        for u in range(32):
            e = b * 32 + u
            s_ = src_ref[e]
            d_ = dst_ref[e]
            agg_ref[pl.ds(d_, 1), :] += p_ref[pl.ds(s_, 1), :]
            deg_ref[pl.ds(d_, 1), :] += one8
        return carry

    lax.fori_loop(0, CE // 32, body, 0)


def _tc_agg2(src_ref, dst_ref, p_ref, agg_ref):
    @pl.when(pl.program_id(0) == 0)
    def _():
        agg_ref[...] = jnp.zeros_like(agg_ref)

    def body(b, carry):
        for u in range(32):
            e = b * 32 + u
            s_ = src_ref[e]
            d_ = dst_ref[e]
            agg_ref[pl.ds(d_, 1), :] += p_ref[pl.ds(s_, 1), :]
        return carry

    lax.fori_loop(0, CE // 32, body, 0)


def _tc_layer2(agg_ref, deg_ref, q1_ref, wl_ref, wr_ref, b_ref,
               p2_ref, q2_ref):
    rdeg = 1.0 / jnp.maximum(deg_ref[:, 0:1], 1.0)
    h = jnp.maximum(agg_ref[...] * rdeg + q1_ref[...], 0.0)
    p2_ref[...] = jnp.dot(h, wl_ref[...], preferred_element_type=jnp.float32)
    q2_ref[...] = (
        jnp.dot(h, wr_ref[...], preferred_element_type=jnp.float32)
        + b_ref[0:1, :]
    )


def _tc_softmax(agg_ref, deg_ref, q2_ref, o_ref):
    rdeg = 1.0 / jnp.maximum(deg_ref[:, 0:1], 1.0)
    s = agg_ref[...] * rdeg + q2_ref[...]
    m = jnp.max(s, axis=1, keepdims=True)
    e = jnp.exp(s - m)
    o_ref[...] = e / jnp.sum(e, axis=1, keepdims=True)


def kernel(x, edge_index, Wl1, bl1, Wr1, Wl2, bl2, Wr2):
    f32 = jnp.float32
    pad_r = jnp.arange(EP - E, dtype=jnp.int32)
    src = jnp.concatenate([edge_index[0], pad_r % N])
    dst = jnp.concatenate([edge_index[1], N + pad_r % (NP - N)])
    xp = jnp.pad(x, ((0, NP - N), (0, 0)))
    bl1b = jnp.broadcast_to(bl1, (8, H))
    bl2b = jnp.broadcast_to(bl2, (8, C))

    p1, q1 = pl.pallas_call(
        _tc_proj1,
        grid=(NB,),
        in_specs=[
            pl.BlockSpec((BN, D), lambda i: (i, 0)),
            pl.BlockSpec((D, H), lambda i: (0, 0)),
            pl.BlockSpec((D, H), lambda i: (0, 0)),
            pl.BlockSpec((8, H), lambda i: (0, 0)),
        ],
        out_specs=[
            pl.BlockSpec((BN, H), lambda i: (i, 0)),
            pl.BlockSpec((BN, H), lambda i: (i, 0)),
        ],
        out_shape=[
            jax.ShapeDtypeStruct((NP, H), f32),
            jax.ShapeDtypeStruct((NP, H), f32),
        ],
    )(xp, Wl1, Wr1, bl1b)

    agg1, deg8 = pl.pallas_call(
        _tc_agg,
        grid=(NG,),
        in_specs=[
            pl.BlockSpec((CE,), lambda i: (i,), memory_space=pltpu.SMEM),
            pl.BlockSpec((CE,), lambda i: (i,), memory_space=pltpu.SMEM),
            pl.BlockSpec((NP, H), lambda i: (0, 0)),
        ],
        out_specs=[
            pl.BlockSpec((NP, H), lambda i: (0, 0)),
            pl.BlockSpec((NP, 8), lambda i: (0, 0)),
        ],
        out_shape=[
            jax.ShapeDtypeStruct((NP, H), f32),
            jax.ShapeDtypeStruct((NP, 8), f32),
        ],
    )(src, dst, p1)

    p2, q2 = pl.pallas_call(
        _tc_layer2,
        grid=(NB,),
        in_specs=[
            pl.BlockSpec((BN, H), lambda i: (i, 0)),
            pl.BlockSpec((BN, 8), lambda i: (i, 0)),
            pl.BlockSpec((BN, H), lambda i: (i, 0)),
            pl.BlockSpec((H, C), lambda i: (0, 0)),
            pl.BlockSpec((H, C), lambda i: (0, 0)),
            pl.BlockSpec((8, C), lambda i: (0, 0)),
        ],
        out_specs=[
            pl.BlockSpec((BN, C), lambda i: (i, 0)),
            pl.BlockSpec((BN, C), lambda i: (i, 0)),
        ],
        out_shape=[
            jax.ShapeDtypeStruct((NP, C), f32),
            jax.ShapeDtypeStruct((NP, C), f32),
        ],
    )(agg1, deg8, q1, Wl2, Wr2, bl2b)

    agg2 = pl.pallas_call(
        _tc_agg2,
        grid=(NG,),
        in_specs=[
            pl.BlockSpec((CE,), lambda i: (i,), memory_space=pltpu.SMEM),
            pl.BlockSpec((CE,), lambda i: (i,), memory_space=pltpu.SMEM),
            pl.BlockSpec((NP, C), lambda i: (0, 0)),
        ],
        out_specs=pl.BlockSpec((NP, C), lambda i: (0, 0)),
        out_shape=jax.ShapeDtypeStruct((NP, C), f32),
    )(src, dst, p2)

    out = pl.pallas_call(
        _tc_softmax,
        grid=(NB,),
        in_specs=[
            pl.BlockSpec((BN, C), lambda i: (i, 0)),
            pl.BlockSpec((BN, 8), lambda i: (i, 0)),
            pl.BlockSpec((BN, C), lambda i: (i, 0)),
        ],
        out_specs=pl.BlockSpec((BN, C), lambda i: (i, 0)),
        out_shape=jax.ShapeDtypeStruct((NP, C), f32),
    )(agg2, deg8, q2)
    return out[:N]
